# Initial kernel scaffold; baseline (speedup 1.0000x reference)
import jax
import jax.numpy as jnp
from jax.experimental import pallas as pl


def kernel(pred_class_logits, pred_compo_logits, compo_co_occurrence_prob, compo_chinese_matrix):
    raise NotImplementedError("write your pallas kernel here")


# speedup vs baseline: 55.3171x; 55.3171x over previous
import functools

import jax
import jax.numpy as jnp
from jax import lax
from jax.experimental import pallas as pl
from jax.experimental.pallas import tpu as pltpu

_THRESH = 0.8
_SCALE = 0.1
_NEG = float("-inf")


def _top5_body(x_ref, vals_ref, idx_ref, *, n):
    x = x_ref[...]
    col = lax.broadcasted_iota(jnp.int32, x.shape, 1)
    x = jnp.where(col < n, x, _NEG)
    m0 = jnp.max(x, axis=1, keepdims=True)
    sumexp = jnp.sum(jnp.exp(x - m0), axis=1, keepdims=True)
    work = x
    vals, idxs = [], []
    for _ in range(5):
        m = jnp.max(work, axis=1, keepdims=True)
        i = jnp.min(jnp.where(work == m, col, n), axis=1, keepdims=True)
        vals.append(m)
        idxs.append(i)
        work = jnp.where(col == i, _NEG, work)
    v = jnp.concatenate(vals, axis=1)
    vals_ref[...] = jnp.exp(v - m0) / sumexp
    idx_ref[...] = jnp.concatenate(idxs, axis=1)


def _compo_body(logit_ref, co_ref, out_ref):
    cs = jax.nn.sigmoid(logit_ref[...])
    sel = jnp.where(cs > _THRESH, cs, 0.0)
    adj = lax.dot_general(
        sel, co_ref[...], (((1,), (0,)), ((), ())),
        precision=lax.Precision.HIGHEST,
        preferred_element_type=jnp.float32,
    )
    out_ref[...] = cs + adj * _SCALE


def _select_body(score_ref, mat_ref, idx_out, sco_out,
                 hit_ref, prev_ref, row_ref, sem, *, c, rs):
    s = score_ref[0]
    ci = (lax.broadcasted_iota(jnp.int32, s.shape, 0) * 128
          + lax.broadcasted_iota(jnp.int32, s.shape, 1))
    max_val = jnp.max(s)
    max_idx = jnp.min(jnp.where(s == max_val, ci, c))
    sel_mask = s > _THRESH
    num_sel = jnp.sum(sel_mask.astype(jnp.int32))
    masked = jnp.where(sel_mask, s, _NEG)
    work0 = jnp.where(ci == max_idx, _NEG, masked)
    s0 = jnp.where(max_val > _THRESH, max_val, _NEG)

    cp = pltpu.make_async_copy(mat_ref.at[max_idx], hit_ref, sem)
    cp.start()
    cp.wait()
    prev_ref[...] = hit_ref[...]
    cnt0 = jnp.sum(hit_ref[...])

    def cond(carry):
        i, cnt, _work, _vl, _vp = carry
        return (i < num_sel) & (cnt > 1)

    def body(carry):
        i, cnt, work, vl, _vp = carry
        v = jnp.max(work)
        o = jnp.min(jnp.where(work == v, ci, c))
        cp2 = pltpu.make_async_copy(mat_ref.at[o], row_ref, sem)
        cp2.start()
        cp2.wait()
        h = hit_ref[...]
        prev_ref[...] = h
        nh = h * row_ref[...]
        hit_ref[...] = nh
        return (i + jnp.int32(1), jnp.sum(nh),
                jnp.where(ci == o, _NEG, work), v, vl)

    i, cnt, _, v_last, v_prev = lax.while_loop(
        cond, body, (jnp.int32(1), cnt0, work0, s0, s0))

    bt = cnt == 0
    i_f = jnp.where(bt, i - 1, i)
    hit = jnp.where(bt, prev_ref[...], hit_ref[...])
    val = jnp.where(bt, v_prev, v_last)
    hit_score = jnp.where(i_f == 1, max_val, val)
    n_idx = jnp.sum(hit)

    fi = (lax.broadcasted_iota(jnp.int32, hit.shape, 0) * rs
          + lax.broadcasted_iota(jnp.int32, hit.shape, 1))
    big = jnp.int32(2 ** 30)
    cand = jnp.where(hit == 1, fi, big)
    lane = lax.broadcasted_iota(jnp.int32, (1, 128), 1)
    vec = jnp.full((1, 128), -1, jnp.int32)
    for j in range(5):
        m = jnp.min(cand)
        vj = jnp.where(n_idx > j, m, jnp.int32(-1))
        vec = jnp.where(lane == j, vj, vec)
        cand = jnp.where(cand == m, big, cand)
    idx_out[0] = vec
    sco_out[0] = jnp.full((1, 128), hit_score, jnp.float32)


def kernel(pred_class_logits, pred_compo_logits,
           compo_co_occurrence_prob, compo_chinese_matrix):
    b, n_chars = pred_class_logits.shape
    n_compo = pred_compo_logits.shape[1]
    rb = 8

    class_scores, class_indices = pl.pallas_call(
        functools.partial(_top5_body, n=n_chars),
        grid=(b // rb,),
        in_specs=[pl.BlockSpec((rb, n_chars), lambda i: (i, 0))],
        out_specs=[pl.BlockSpec((rb, 5), lambda i: (i, 0)),
                   pl.BlockSpec((rb, 5), lambda i: (i, 0))],
        out_shape=[jax.ShapeDtypeStruct((b, 5), jnp.float32),
                   jax.ShapeDtypeStruct((b, 5), jnp.int32)],
    )(pred_class_logits)

    scores = pl.pallas_call(
        _compo_body,
        out_shape=jax.ShapeDtypeStruct((b, n_compo), jnp.float32),
    )(pred_compo_logits, compo_co_occurrence_prob)

    mr = 8
    rs = n_chars // mr
    mat3 = compo_chinese_matrix.reshape(n_compo, mr, rs)
    s3 = scores.reshape(b, n_compo // 128, 128)

    idx_o, sco_o = pl.pallas_call(
        functools.partial(_select_body, c=n_compo, rs=rs),
        grid=(b,),
        in_specs=[
            pl.BlockSpec((1, n_compo // 128, 128), lambda i: (i, 0, 0)),
            pl.BlockSpec(memory_space=pl.ANY),
        ],
        out_specs=[pl.BlockSpec((1, 1, 128), lambda i: (i, 0, 0)),
                   pl.BlockSpec((1, 1, 128), lambda i: (i, 0, 0))],
        out_shape=[jax.ShapeDtypeStruct((b, 1, 128), jnp.int32),
                   jax.ShapeDtypeStruct((b, 1, 128), jnp.float32)],
        scratch_shapes=[pltpu.VMEM((mr, rs), jnp.int32),
                        pltpu.VMEM((mr, rs), jnp.int32),
                        pltpu.VMEM((mr, rs), jnp.int32),
                        pltpu.SemaphoreType.DMA],
    )(s3, mat3)

    hit_idx = idx_o[:, 0, :5]
    hit_scores = sco_o[:, 0, 0]
    num_compo_hit = jnp.sum((hit_idx != -1).astype(jnp.int32), axis=1)
    c0 = class_indices[:, 0].astype(jnp.int32)
    combined_pred1 = jnp.where(num_compo_hit == 1, hit_idx[:, 0], c0)
    combined_pred2 = jnp.where(
        (class_scores[:, 0] < 0.85) & (num_compo_hit == 1), hit_idx[:, 0], c0)
    return (class_indices, class_scores, hit_idx, hit_scores,
            combined_pred1, combined_pred2)


# speedup vs baseline: 173.0008x; 3.1274x over previous
import functools

import jax
import jax.numpy as jnp
from jax import lax
from jax.experimental import pallas as pl
from jax.experimental.pallas import tpu as pltpu
from jax.experimental.pallas import tpu_sc as plsc

_THRESH = 0.8
_SCALE = 0.1
_NEG = float("-inf")


def _top5_body(x_ref, vals_ref, idx_ref, *, n):
    x = x_ref[...]
    col = lax.broadcasted_iota(jnp.int32, x.shape, 1)
    x = jnp.where(col < n, x, _NEG)
    m0 = jnp.max(x, axis=1, keepdims=True)
    sumexp = jnp.sum(jnp.exp(x - m0), axis=1, keepdims=True)
    work = x
    vals, idxs = [], []
    for _ in range(5):
        m = jnp.max(work, axis=1, keepdims=True)
        i = jnp.min(jnp.where(work == m, col, n), axis=1, keepdims=True)
        vals.append(m)
        idxs.append(i)
        work = jnp.where(col == i, _NEG, work)
    v = jnp.concatenate(vals, axis=1)
    vals_ref[...] = jnp.exp(v - m0) / sumexp
    idx_ref[...] = jnp.concatenate(idxs, axis=1)


def _compo_body(logit_ref, co_ref, out_ref):
    cs = jax.nn.sigmoid(logit_ref[...])
    sel = jnp.where(cs > _THRESH, cs, 0.0)
    adj = lax.dot_general(
        sel, co_ref[...], (((1,), (0,)), ((), ())),
        precision=lax.Precision.HIGHEST,
        preferred_element_type=jnp.float32,
    )
    out_ref[...] = cs + adj * _SCALE


def _sc_select(scores, mat):
    b, c = scores.shape
    n = mat.shape[1]
    nw = 32
    spw = b // nw
    nchunk = n // 16
    cchunk = c // 16
    mesh = plsc.VectorSubcoreMesh(core_axis_name="c", subcore_axis_name="s")

    @functools.partial(
        pl.kernel,
        out_type=[jax.ShapeDtypeStruct((b, 16), jnp.int32),
                  jax.ShapeDtypeStruct((b, 16), jnp.float32)],
        mesh=mesh,
        scratch_types=[
            pltpu.VMEM((1, c), jnp.float32),
            pltpu.VMEM((1, n), jnp.int32),
            pltpu.VMEM((1, n), jnp.int32),
            pltpu.VMEM((1, n), jnp.int32),
            pltpu.VMEM((n + 16,), jnp.int32),
            pltpu.VMEM((1, 16), jnp.int32),
            pltpu.VMEM((1, 16), jnp.float32),
            pltpu.SemaphoreType.DMA,
            pltpu.SemaphoreType.DMA,
        ],
        compiler_params=pltpu.CompilerParams(needs_layout_passes=False),
    )
    def sel(scores_hbm, mat_hbm, idx_hbm, sco_hbm,
            w_ref, hit_ref, prev_ref, row_ref, ol_ref, hv_ref, sv_ref,
            sem1, sem2):
        wid = lax.axis_index("s") * 2 + lax.axis_index("c")
        lanes = lax.iota(jnp.int32, 16)
        neg = jnp.full((16,), _NEG, jnp.float32)
        zi = jnp.zeros((16,), jnp.int32)
        onei = jnp.full((16,), 1, jnp.int32)

        def sample_body(s_i, _):
            bidx = wid * spw + s_i
            pltpu.sync_copy(scores_hbm.at[pl.ds(bidx, 1)], w_ref)

            def spass(k, carry):
                bestv, besti, csel = carry
                v = w_ref[0, pl.ds(k * 16, 16)]
                idxv = lanes + k * 16
                upd = v > bestv
                bestv = jnp.where(upd, v, bestv)
                besti = jnp.where(upd, idxv, besti)
                csel = csel + jnp.where(v > _THRESH, onei, zi)
                w_ref[0, pl.ds(k * 16, 16)] = jnp.where(v > _THRESH, v, neg)
                return bestv, besti, csel

            bestv, besti, csel = lax.fori_loop(0, cchunk, spass, (neg, zi, zi))
            gmax = jnp.max(bestv)
            max_idx = jnp.min(jnp.where(bestv == gmax, besti, c))
            num_sel = jnp.sum(csel)
            s0 = jnp.where(gmax > _THRESH, gmax, _NEG)

            ch0 = (max_idx // 16) * 16
            l0 = max_idx % 16
            v0 = w_ref[0, pl.ds(ch0, 16)]
            w_ref[0, pl.ds(ch0, 16)] = jnp.where(lanes == l0, neg, v0)

            pltpu.async_copy(mat_hbm.at[pl.ds(max_idx, 1)], hit_ref, sem1).wait()

            def cpass(k, acc):
                return acc + hit_ref[0, pl.ds(k * 16, 16)]

            cnt0 = jnp.sum(lax.fori_loop(0, nchunk, cpass, zi))

            def wcond(carry):
                i, cnt, _cp, _vl, _vp = carry
                return (i < num_sel) & (cnt > 1)

            def wbody(carry):
                i, cnt, _cp, vl, _vp = carry

                def apass(k, c2):
                    bv, bi = c2
                    v = w_ref[0, pl.ds(k * 16, 16)]
                    idxv = lanes + k * 16
                    upd = v > bv
                    return jnp.where(upd, v, bv), jnp.where(upd, idxv, bi)

                bv, bi = lax.fori_loop(0, cchunk, apass, (neg, zi))
                v = jnp.max(bv)
                o = jnp.min(jnp.where(bv == v, bi, c))
                cho = (o // 16) * 16
                lo = o % 16
                vv = w_ref[0, pl.ds(cho, 16)]
                w_ref[0, pl.ds(cho, 16)] = jnp.where(lanes == lo, neg, vv)

                pltpu.async_copy(mat_hbm.at[pl.ds(o, 1)], row_ref, sem2).wait()

                def ipass(k, acc):
                    sl = pl.ds(k * 16, 16)
                    h = hit_ref[0, sl]
                    prev_ref[0, sl] = h
                    hn = h * row_ref[0, sl]
                    hit_ref[0, sl] = hn
                    return acc + hn

                cnt_new = jnp.sum(lax.fori_loop(0, nchunk, ipass, zi))
                return (i + jnp.int32(1), cnt_new, cnt, v, vl)

            i, cnt, cntp, v_last, v_prev = lax.while_loop(
                wcond, wbody, (jnp.int32(1), cnt0, cnt0, s0, s0))

            bt = cnt == 0
            btv = jnp.full((16,), bt)
            i_f = jnp.where(bt, i - 1, i)
            n_idx = jnp.where(bt, cntp, cnt)
            val = jnp.where(bt, v_prev, v_last)
            hit_score = jnp.where(i_f == 1, gmax, val)

            def epass(k, off):
                sl = pl.ds(k * 16, 16)
                h = jnp.where(btv, prev_ref[0, sl], hit_ref[0, sl])
                m = h == 1
                idxv = lanes + k * 16
                plsc.store_compressed(ol_ref.at[pl.ds(off, 16)], idxv, mask=m)
                pc = plsc.all_reduce_population_count(m)
                return off + jnp.max(pc)

            lax.fori_loop(0, nchunk, epass, jnp.int32(0))

            ol = ol_ref[pl.ds(0, 16)]
            out5 = jnp.where((lanes < 5) & (lanes < n_idx), ol, jnp.int32(-1))
            hv_ref[0, pl.ds(0, 16)] = out5
            sv_ref[0, pl.ds(0, 16)] = jnp.full((16,), hit_score, jnp.float32)
            pltpu.sync_copy(hv_ref, idx_hbm.at[pl.ds(bidx, 1)])
            pltpu.sync_copy(sv_ref, sco_hbm.at[pl.ds(bidx, 1)])
            return 0

        lax.fori_loop(0, spw, sample_body, 0)

    return sel(scores, mat)


def kernel(pred_class_logits, pred_compo_logits,
           compo_co_occurrence_prob, compo_chinese_matrix):
    b, n_chars = pred_class_logits.shape
    n_compo = pred_compo_logits.shape[1]
    rb = 8

    scores = pl.pallas_call(
        _compo_body,
        out_shape=jax.ShapeDtypeStruct((b, n_compo), jnp.float32),
    )(pred_compo_logits, compo_co_occurrence_prob)

    idx_o, sco_o = _sc_select(scores, compo_chinese_matrix)

    class_scores, class_indices = pl.pallas_call(
        functools.partial(_top5_body, n=n_chars),
        grid=(b // rb,),
        in_specs=[pl.BlockSpec((rb, n_chars), lambda i: (i, 0))],
        out_specs=[pl.BlockSpec((rb, 5), lambda i: (i, 0)),
                   pl.BlockSpec((rb, 5), lambda i: (i, 0))],
        out_shape=[jax.ShapeDtypeStruct((b, 5), jnp.float32),
                   jax.ShapeDtypeStruct((b, 5), jnp.int32)],
    )(pred_class_logits)

    hit_idx = idx_o[:, :5]
    hit_scores = sco_o[:, 0]
    num_compo_hit = jnp.sum((hit_idx != -1).astype(jnp.int32), axis=1)
    c0 = class_indices[:, 0].astype(jnp.int32)
    combined_pred1 = jnp.where(num_compo_hit == 1, hit_idx[:, 0], c0)
    combined_pred2 = jnp.where(
        (class_scores[:, 0] < 0.85) & (num_compo_hit == 1), hit_idx[:, 0], c0)
    return (class_indices, class_scores, hit_idx, hit_scores,
            combined_pred1, combined_pred2)


# speedup vs baseline: 227.1654x; 1.3131x over previous
import functools

import jax
import jax.numpy as jnp
from jax import lax
from jax.experimental import pallas as pl
from jax.experimental.pallas import tpu as pltpu
from jax.experimental.pallas import tpu_sc as plsc

_THRESH = 0.8
_SCALE = 0.1
_NEG = float("-inf")


def _top5_body(x_ref, vals_ref, idx_ref, *, n):
    x = x_ref[...]
    col = lax.broadcasted_iota(jnp.int32, x.shape, 1)
    x = jnp.where(col < n, x, _NEG)
    m0 = jnp.max(x, axis=1, keepdims=True)
    sumexp = jnp.sum(jnp.exp(x - m0), axis=1, keepdims=True)
    work = x
    vals, idxs = [], []
    for _ in range(5):
        m = jnp.max(work, axis=1, keepdims=True)
        i = jnp.min(jnp.where(work == m, col, n), axis=1, keepdims=True)
        vals.append(m)
        idxs.append(i)
        work = jnp.where(col == i, _NEG, work)
    v = jnp.concatenate(vals, axis=1)
    vals_ref[...] = jnp.exp(v - m0) / sumexp
    idx_ref[...] = jnp.concatenate(idxs, axis=1)


def _compo_body(logit_ref, co_ref, out_ref):
    cs = jax.nn.sigmoid(logit_ref[...])
    sel = jnp.where(cs > _THRESH, cs, 0.0)
    adj = lax.dot_general(
        sel, co_ref[...], (((1,), (0,)), ((), ())),
        precision=lax.Precision.HIGHEST,
        preferred_element_type=jnp.float32,
    )
    out_ref[...] = cs + adj * _SCALE


def _sc_select(scores, mat):
    b, c = scores.shape
    n = mat.shape[1]
    nw = 32
    spw = b // nw
    nchunk = n // 16
    cchunk = c // 16
    mesh = plsc.VectorSubcoreMesh(core_axis_name="c", subcore_axis_name="s")

    @functools.partial(
        pl.kernel,
        out_type=[jax.ShapeDtypeStruct((b, 16), jnp.int32),
                  jax.ShapeDtypeStruct((b, 16), jnp.float32)],
        mesh=mesh,
        scratch_types=[
            pltpu.VMEM((1, c), jnp.float32),
            pltpu.VMEM((1, n), jnp.int32),
            pltpu.VMEM((1, n), jnp.int32),
            pltpu.VMEM((n + 16,), jnp.int32),
            pltpu.VMEM((n + 16,), jnp.int32),
            pltpu.VMEM((1, 16), jnp.int32),
            pltpu.VMEM((1, 16), jnp.float32),
            pltpu.SemaphoreType.DMA,
            pltpu.SemaphoreType.DMA,
        ],
        compiler_params=pltpu.CompilerParams(needs_layout_passes=False),
    )
    def sel(scores_hbm, mat_hbm, idx_hbm, sco_hbm,
            w_ref, rowa_ref, rowb_ref, lcur_ref, lprev_ref, hv_ref, sv_ref,
            sem1, sem2):
        wid = lax.axis_index("s") * 2 + lax.axis_index("c")
        lanes = lax.iota(jnp.int32, 16)
        neg = jnp.full((16,), _NEG, jnp.float32)
        zi = jnp.zeros((16,), jnp.int32)
        onei = jnp.full((16,), 1, jnp.int32)

        def sample_body(s_i, _):
            bidx = wid * spw + s_i
            pltpu.sync_copy(scores_hbm.at[pl.ds(bidx, 1)], w_ref)

            def spass(k, carry):
                bestv, besti, csel = carry
                v = w_ref[0, pl.ds(k * 16, 16)]
                idxv = lanes + k * 16
                upd = v > bestv
                bestv = jnp.where(upd, v, bestv)
                besti = jnp.where(upd, idxv, besti)
                csel = csel + jnp.where(v > _THRESH, onei, zi)
                w_ref[0, pl.ds(k * 16, 16)] = jnp.where(v > _THRESH, v, neg)
                return bestv, besti, csel

            bestv, besti, csel = lax.fori_loop(0, cchunk, spass, (neg, zi, zi))
            gmax = jnp.max(bestv)
            max_idx = jnp.min(jnp.where(bestv == gmax, besti, c))
            num_sel = jnp.sum(csel)
            s0 = jnp.where(gmax > _THRESH, gmax, _NEG)

            ch0 = (max_idx // 16) * 16
            l0 = max_idx % 16
            v0 = w_ref[0, pl.ds(ch0, 16)]
            w_ref[0, pl.ds(ch0, 16)] = jnp.where(lanes == l0, neg, v0)
            dma0 = pltpu.async_copy(mat_hbm.at[pl.ds(max_idx, 1)], rowa_ref, sem1)

            def apass(k, c2):
                bv, bi = c2
                v = w_ref[0, pl.ds(k * 16, 16)]
                idxv = lanes + k * 16
                upd = v > bv
                return jnp.where(upd, v, bv), jnp.where(upd, idxv, bi)

            bv1, bi1 = lax.fori_loop(0, cchunk, apass, (neg, zi))
            v1 = jnp.max(bv1)
            o1 = jnp.min(jnp.where(bv1 == v1, bi1, c))
            ch1 = (o1 // 16) * 16
            l1 = o1 % 16
            vv1 = w_ref[0, pl.ds(ch1, 16)]
            w_ref[0, pl.ds(ch1, 16)] = jnp.where(lanes == l1, neg, vv1)
            dma1 = pltpu.async_copy(mat_hbm.at[pl.ds(o1, 1)], rowb_ref, sem2)
            dma0.wait()

            def epass(k, off):
                v = rowa_ref[0, pl.ds(k * 16, 16)]
                m = v == 1
                idxv = lanes + k * 16
                plsc.store_compressed(lcur_ref.at[pl.ds(off, 16)], idxv, mask=m)
                pc = plsc.all_reduce_population_count(m)
                return off + jnp.max(pc)

            cnt0 = lax.fori_loop(0, nchunk, epass, jnp.int32(0))

            def gather_round(cnt):
                def gpass(j, off):
                    pos = lcur_ref[pl.ds(j * 16, 16)]
                    lprev_ref[pl.ds(j * 16, 16)] = pos
                    valid = (lanes + j * 16) < cnt
                    pos_s = jnp.where(valid, pos, zi)
                    vals = plsc.load_gather(rowb_ref, [zi, pos_s], mask=valid)
                    m = valid & (vals == 1)
                    plsc.store_compressed(lcur_ref.at[pl.ds(off, 16)], pos_s,
                                          mask=m)
                    pc = plsc.all_reduce_population_count(m)
                    return off + jnp.max(pc)

                nch = (cnt + 15) // 16
                return lax.fori_loop(0, nch, gpass, jnp.int32(0))

            def wcond(carry):
                i, cnt, _cp, _vl, _vp, _f = carry
                return (i < num_sel) & (cnt > 1)

            def wbody(carry):
                i, cnt, _cp, vl, _vp, first = carry

                def later_round(_):
                    bv, bi = lax.fori_loop(0, cchunk, apass, (neg, zi))
                    v = jnp.max(bv)
                    o = jnp.min(jnp.where(bv == v, bi, c))
                    cho = (o // 16) * 16
                    lo = o % 16
                    vv = w_ref[0, pl.ds(cho, 16)]
                    w_ref[0, pl.ds(cho, 16)] = jnp.where(lanes == lo, neg, vv)
                    pltpu.async_copy(mat_hbm.at[pl.ds(o, 1)], rowb_ref,
                                     sem2).wait()
                    return v

                def first_round(_):
                    dma1.wait()
                    return v1

                v = lax.cond(first, first_round, later_round, 0)
                cnt_new = gather_round(cnt)
                return (i + jnp.int32(1), cnt_new, cnt, v, vl,
                        jnp.bool_(False))

            i, cnt, cntp, v_last, v_prev, first = lax.while_loop(
                wcond, wbody,
                (jnp.int32(1), cnt0, cnt0, s0, s0, jnp.bool_(True)))

            @pl.when(first)
            def _():
                dma1.wait()

            bt = cnt == 0
            btv = jnp.full((16,), bt)
            i_f = jnp.where(bt, i - 1, i)
            n_idx = jnp.where(bt, cntp, cnt)
            val = jnp.where(bt, v_prev, v_last)
            hit_score = jnp.where(i_f == 1, gmax, val)

            lsrc = jnp.where(btv, lprev_ref[pl.ds(0, 16)],
                             lcur_ref[pl.ds(0, 16)])
            out5 = jnp.where((lanes < 5) & (lanes < n_idx), lsrc, jnp.int32(-1))
            hv_ref[0, pl.ds(0, 16)] = out5
            sv_ref[0, pl.ds(0, 16)] = jnp.full((16,), hit_score, jnp.float32)
            pltpu.sync_copy(hv_ref, idx_hbm.at[pl.ds(bidx, 1)])
            pltpu.sync_copy(sv_ref, sco_hbm.at[pl.ds(bidx, 1)])
            return 0

        lax.fori_loop(0, spw, sample_body, 0)

    return sel(scores, mat)


def kernel(pred_class_logits, pred_compo_logits,
           compo_co_occurrence_prob, compo_chinese_matrix):
    b, n_chars = pred_class_logits.shape
    n_compo = pred_compo_logits.shape[1]
    rb = 8

    scores = pl.pallas_call(
        _compo_body,
        out_shape=jax.ShapeDtypeStruct((b, n_compo), jnp.float32),
    )(pred_compo_logits, compo_co_occurrence_prob)

    idx_o, sco_o = _sc_select(scores, compo_chinese_matrix)

    class_scores, class_indices = pl.pallas_call(
        functools.partial(_top5_body, n=n_chars),
        grid=(b // rb,),
        in_specs=[pl.BlockSpec((rb, n_chars), lambda i: (i, 0))],
        out_specs=[pl.BlockSpec((rb, 5), lambda i: (i, 0)),
                   pl.BlockSpec((rb, 5), lambda i: (i, 0))],
        out_shape=[jax.ShapeDtypeStruct((b, 5), jnp.float32),
                   jax.ShapeDtypeStruct((b, 5), jnp.int32)],
    )(pred_class_logits)

    hit_idx = idx_o[:, :5]
    hit_scores = sco_o[:, 0]
    num_compo_hit = jnp.sum((hit_idx != -1).astype(jnp.int32), axis=1)
    c0 = class_indices[:, 0].astype(jnp.int32)
    combined_pred1 = jnp.where(num_compo_hit == 1, hit_idx[:, 0], c0)
    combined_pred2 = jnp.where(
        (class_scores[:, 0] < 0.85) & (num_compo_hit == 1), hit_idx[:, 0], c0)
    return (class_indices, class_scores, hit_idx, hit_scores,
            combined_pred1, combined_pred2)
